# baseline (speedup 1.0000x reference)
import jax
import jax.numpy as jnp
from jax import lax
from jax.experimental import pallas as pl
from jax.experimental.pallas import tpu as pltpu
from jax.experimental.pallas import tpu_sc as plsc

_NC = 2
_NS = 16
_NW = _NC * _NS
_K = 128


def _mm_body(x_ref, w_ref, b_ref, o_ref):
    o_ref[...] = (
        jnp.dot(x_ref[...], w_ref[...], preferred_element_type=jnp.float32)
        + b_ref[...]
    )


def _matmul_bias(x, w, b, block_rows=512):
    n, d_in = x.shape
    d_out = w.shape[1]
    return pl.pallas_call(
        _mm_body,
        grid=(pl.cdiv(n, block_rows),),
        in_specs=[
            pl.BlockSpec((block_rows, d_in), lambda i: (i, 0)),
            pl.BlockSpec((d_in, d_out), lambda i: (0, 0)),
            pl.BlockSpec((1, d_out), lambda i: (0, 0)),
        ],
        out_specs=pl.BlockSpec((block_rows, d_out), lambda i: (i, 0)),
        out_shape=jax.ShapeDtypeStruct((n, d_out), jnp.float32),
    )(x, w, b.reshape(1, d_out))


def _norm_mm_body(a0_ref, a1_ref, d0_ref, d1_ref, w_ref, b_ref, o_ref):
    deg = jnp.maximum(d0_ref[...] + d1_ref[...], 1.0)
    h = jnp.maximum((a0_ref[...] + a1_ref[...]) / deg, 0.0)
    o_ref[...] = (
        jnp.dot(h, w_ref[...], preferred_element_type=jnp.float32) + b_ref[...]
    )


def _norm_relu_matmul(a0, a1, d0, d1, w, b, block_rows=512):
    n, d_in = a0.shape
    d_out = w.shape[1]
    return pl.pallas_call(
        _norm_mm_body,
        grid=(pl.cdiv(n, block_rows),),
        in_specs=[
            pl.BlockSpec((block_rows, d_in), lambda i: (i, 0)),
            pl.BlockSpec((block_rows, d_in), lambda i: (i, 0)),
            pl.BlockSpec((block_rows, 1), lambda i: (i, 0)),
            pl.BlockSpec((block_rows, 1), lambda i: (i, 0)),
            pl.BlockSpec((d_in, d_out), lambda i: (0, 0)),
            pl.BlockSpec((1, d_out), lambda i: (0, 0)),
        ],
        out_specs=pl.BlockSpec((block_rows, d_out), lambda i: (i, 0)),
        out_shape=jax.ShapeDtypeStruct((n, d_out), jnp.float32),
    )(a0, a1, d0, d1, w, b.reshape(1, d_out))


def _norm_body(a0_ref, a1_ref, d0_ref, d1_ref, o_ref):
    deg = jnp.maximum(d0_ref[...] + d1_ref[...], 1.0)
    o_ref[...] = (a0_ref[...] + a1_ref[...]) / deg


def _norm(a0, a1, d0, d1, block_rows=512):
    n, d = a0.shape
    return pl.pallas_call(
        _norm_body,
        grid=(pl.cdiv(n, block_rows),),
        in_specs=[
            pl.BlockSpec((block_rows, d), lambda i: (i, 0)),
            pl.BlockSpec((block_rows, d), lambda i: (i, 0)),
            pl.BlockSpec((block_rows, 1), lambda i: (i, 0)),
            pl.BlockSpec((block_rows, 1), lambda i: (i, 0)),
        ],
        out_specs=pl.BlockSpec((block_rows, d), lambda i: (i, 0)),
        out_shape=jax.ShapeDtypeStruct((n, d), jnp.float32),
    )(a0, a1, d0, d1)


def _sc_aggregate(h2, idx_packed, z_nd):
    npad, d = h2.shape
    n_chunks = idx_packed.shape[1]
    rows_per_tile = z_nd.shape[0]
    n = rows_per_tile * _NS
    mesh = plsc.VectorSubcoreMesh(core_axis_name="c", subcore_axis_name="s")

    out_type = [jax.ShapeDtypeStruct((_NC, _NS, rows_per_tile, d), jnp.float32)]
    scratch = [
        pltpu.VMEM_SHARED((npad, d), jnp.float32),
        pltpu.VMEM((2, _K), jnp.int32),
        pltpu.VMEM((2, _K), jnp.int32),
        pltpu.VMEM((_K, d), jnp.float32),
        pltpu.VMEM((_K, d), jnp.float32),
        pltpu.SemaphoreType.DMA,
        pltpu.SemaphoreType.DMA,
        pltpu.SemaphoreType.DMA,
        pltpu.SemaphoreType.DMA,
    ]

    def body(h2_ref, idx_ref, z_nd_ref, acc_out,
             acc_sh, idx0, idx1, rows0, rows1, gsem0, gsem1, isem0, isem1):
        c = lax.axis_index("c")
        s = lax.axis_index("s")
        wid = c * _NS + s
        row0 = s * rows_per_tile
        pltpu.sync_copy(z_nd_ref, acc_sh.at[pl.ds(row0, rows_per_tile)])
        plsc.subcore_barrier()

        pltpu.sync_copy(idx_ref.at[wid, 0], idx0)
        pltpu.sync_copy(idx_ref.at[wid, 1], idx1)
        pltpu.async_copy(h2_ref.at[idx0.at[0]], rows0, gsem0)

        def pair(gg, carry):
            j0 = 2 * gg
            j1 = j0 + 1
            pltpu.make_async_copy(h2_ref.at[idx0.at[0]], rows0, gsem0).wait()

            @pl.when(gg > 0)
            def _():
                pltpu.make_async_copy(
                    idx_ref.at[wid, j1], idx1, isem1).wait()

            pltpu.async_copy(h2_ref.at[idx1.at[0]], rows1, gsem1)
            pltpu.sync_copy(rows0, acc_sh.at[idx0.at[1]], add=True)

            @pl.when(j0 + 2 < n_chunks)
            def _():
                pltpu.async_copy(idx_ref.at[wid, j0 + 2], idx0, isem0)

            pltpu.make_async_copy(h2_ref.at[idx1.at[0]], rows1, gsem1).wait()

            @pl.when(j0 + 2 < n_chunks)
            def _():
                pltpu.make_async_copy(
                    idx_ref.at[wid, j0 + 2], idx0, isem0).wait()
                pltpu.async_copy(h2_ref.at[idx0.at[0]], rows0, gsem0)

            pltpu.sync_copy(rows1, acc_sh.at[idx1.at[1]], add=True)

            @pl.when(j1 + 2 < n_chunks)
            def _():
                pltpu.async_copy(idx_ref.at[wid, j1 + 2], idx1, isem1)

            return carry

        lax.fori_loop(0, n_chunks // 2, pair, 0)
        if n_chunks % 2 == 1:
            pltpu.make_async_copy(h2_ref.at[idx0.at[0]], rows0, gsem0).wait()
            pltpu.sync_copy(rows0, acc_sh.at[idx0.at[1]], add=True)
        plsc.subcore_barrier()
        pltpu.sync_copy(acc_sh.at[pl.ds(row0, rows_per_tile)],
                        acc_out.at[c, s])

    f = pl.kernel(body, out_type=out_type, mesh=mesh, scratch_types=scratch)
    (acc,) = f(h2, idx_packed, z_nd)
    return acc.reshape(_NC, n, d)


def _sc_degree(idx_packed, npad, z_nd, ones_kd):
    n_chunks = idx_packed.shape[1]
    d = z_nd.shape[1]
    rows_per_tile = z_nd.shape[0]
    n = rows_per_tile * _NS
    mesh = plsc.VectorSubcoreMesh(core_axis_name="c", subcore_axis_name="s")

    out_type = [jax.ShapeDtypeStruct((_NC, _NS, rows_per_tile, d), jnp.float32)]
    scratch = [
        pltpu.VMEM_SHARED((npad, d), jnp.float32),
        pltpu.VMEM((2, _K), jnp.int32),
        pltpu.VMEM((2, _K), jnp.int32),
        pltpu.VMEM((_K, d), jnp.float32),
        pltpu.SemaphoreType.DMA,
        pltpu.SemaphoreType.DMA,
    ]

    def body(idx_ref, z_nd_ref, ones_ref, deg_out,
             deg_sh, idx0, idx1, ones_v, isem0, isem1):
        c = lax.axis_index("c")
        s = lax.axis_index("s")
        wid = c * _NS + s
        row0 = s * rows_per_tile
        pltpu.sync_copy(z_nd_ref, deg_sh.at[pl.ds(row0, rows_per_tile)])
        pltpu.sync_copy(ones_ref, ones_v)
        pltpu.sync_copy(idx_ref.at[wid, 0], idx0)
        pltpu.sync_copy(idx_ref.at[wid, 1], idx1)
        plsc.subcore_barrier()

        def pair(gg, carry):
            j0 = 2 * gg
            j1 = j0 + 1

            @pl.when(gg > 0)
            def _():
                pltpu.make_async_copy(
                    idx_ref.at[wid, j0], idx0, isem0).wait()

            pltpu.sync_copy(ones_v, deg_sh.at[idx0.at[1]], add=True)

            @pl.when(j0 + 2 < n_chunks)
            def _():
                pltpu.async_copy(idx_ref.at[wid, j0 + 2], idx0, isem0)

            @pl.when(gg > 0)
            def _():
                pltpu.make_async_copy(
                    idx_ref.at[wid, j1], idx1, isem1).wait()

            pltpu.sync_copy(ones_v, deg_sh.at[idx1.at[1]], add=True)

            @pl.when(j1 + 2 < n_chunks)
            def _():
                pltpu.async_copy(idx_ref.at[wid, j1 + 2], idx1, isem1)

            return carry

        lax.fori_loop(0, n_chunks // 2, pair, 0)
        if n_chunks % 2 == 1:
            pltpu.make_async_copy(
                idx_ref.at[wid, n_chunks - 1], idx0, isem0).wait()
            pltpu.sync_copy(ones_v, deg_sh.at[idx0.at[1]], add=True)
        plsc.subcore_barrier()
        pltpu.sync_copy(deg_sh.at[pl.ds(row0, rows_per_tile)],
                        deg_out.at[c, s])

    f = pl.kernel(body, out_type=out_type, mesh=mesh, scratch_types=scratch)
    (deg,) = f(idx_packed, z_nd, ones_kd)
    return deg.reshape(_NC, n, d)


def kernel(node_presentation, edges, W1, b1, W2, b2):
    x = node_presentation
    n, _ = x.shape
    e = edges.shape[0]
    assert n % _NS == 0
    n_chunks = -(-e // (_NW * _K))
    e_pad = _NW * n_chunks * _K
    npad = n + 16

    src = edges[:, 0].astype(jnp.int32)
    dst = edges[:, 1].astype(jnp.int32)
    if e_pad > e:
        src = jnp.concatenate([src, jnp.zeros((e_pad - e,), jnp.int32)])
        dst = jnp.concatenate([dst, jnp.full((e_pad - e,), n, jnp.int32)])
    idx_packed = jnp.stack(
        [src.reshape(_NW, n_chunks, _K), dst.reshape(_NW, n_chunks, _K)],
        axis=2)

    d_hid = W1.shape[1]
    z_nd = jnp.zeros((n // _NS, d_hid), jnp.float32)
    ones_kd = jnp.ones((_K, d_hid), jnp.float32)

    h2 = _matmul_bias(x, W1, b1)
    h2 = jnp.concatenate([h2, jnp.zeros((npad - n, d_hid), jnp.float32)])
    acc1 = _sc_aggregate(h2, idx_packed, z_nd)
    deg = _sc_degree(idx_packed, npad, z_nd, ones_kd)
    d0 = deg[0, :, :1]
    d1 = deg[1, :, :1]
    h2b = _norm_relu_matmul(acc1[0], acc1[1], d0, d1, W2, b2)
    h2b = jnp.concatenate([h2b, jnp.zeros((npad - n, d_hid), jnp.float32)])
    acc2 = _sc_aggregate(h2b, idx_packed, z_nd)
    return _norm(acc2[0], acc2[1], d0, d1)
